# baseline (speedup 1.0000x reference)
import functools

import jax
import jax.numpy as jnp
from jax.experimental import pallas as pl
from jax.experimental.pallas import tpu as pltpu

B, N, K = 8, 1024, 40
EPS = 1e-5
BNS = 1.0 / (1.0 + EPS) ** 0.5
RB = 256
NEG = -jnp.inf


def _lrelu(x):
    return jnp.where(x > 0, x, 0.2 * x)


def _knn_body(xt_ref, idx_ref, pd_ref):
    r = pl.program_id(1)
    xt = xt_ref[0]
    rows = xt_ref[0, pl.ds(r * RB, RB), :]
    g = jax.lax.dot_general(rows, xt, (((1,), (1,)), ((), ())),
                            preferred_element_type=jnp.float32)
    xxc = jnp.sum(xt * xt, axis=1)[None, :]
    xxr = jnp.sum(rows * rows, axis=1)[:, None]
    pd_ref[...] = 2.0 * g - xxr - xxc

    iota = jax.lax.broadcasted_iota(jnp.int32, (RB, N), 1)

    sels = []
    for _ in range(K):
        pd = pd_ref[...]
        m = jnp.max(pd, axis=1, keepdims=True)
        cand = jnp.where(pd == m, iota, N)
        sel = jnp.min(cand, axis=1, keepdims=True)
        sels.append(sel)
        pd_ref[...] = jnp.where(iota == sel, NEG, pd)
    idx_ref[0] = jnp.concatenate(sels, axis=1)


def _knn_idx(xt):
    c = xt.shape[-1]
    return pl.pallas_call(
        _knn_body,
        grid=(B, N // RB),
        in_specs=[pl.BlockSpec((1, N, c), lambda b, r: (b, 0, 0))],
        out_specs=pl.BlockSpec((1, RB, K), lambda b, r: (b, r, 0)),
        out_shape=jax.ShapeDtypeStruct((B, N, K), jnp.int32),
        scratch_shapes=[pltpu.VMEM((RB, N), jnp.float32)],
    )(xt)


def _edge_body(xt_ref, wd_ref, wcd_ref, wb_ref, out_ref, su_ref, ut_ref, *,
               has_wb):
    @pl.when((pl.program_id(0) == 0) & (pl.program_id(1) == 0))
    def _build_ut():
        ri = jax.lax.broadcasted_iota(jnp.int32, (N, N), 0)
        ci = jax.lax.broadcasted_iota(jnp.int32, (N, N), 1)
        ut_ref[...] = jnp.where(ri <= ci, 1.0, 0.0)

    r = pl.program_id(1)
    xt = xt_ref[0]
    rows = xt_ref[0, pl.ds(r * RB, RB), :]
    g = jax.lax.dot_general(rows, xt, (((1,), (1,)), ((), ())),
                            preferred_element_type=jnp.float32)
    xxc = jnp.sum(xt * xt, axis=1)[None, :]
    xxr = jnp.sum(rows * rows, axis=1)[:, None]
    pd = 2.0 * g - xxr - xxc
    u = jax.lax.bitcast_convert_type(pd, jnp.int32)
    su = jnp.where(u >= 0, u, u ^ jnp.int32(0x7FFFFFFF))
    su_ref[...] = su

    p = jax.lax.dot_general(xt, wd_ref[...], (((1,), (0,)), ((), ())),
                            preferred_element_type=jnp.float32)
    q = jax.lax.dot_general(rows, wcd_ref[...], (((1,), (0,)), ((), ())),
                            preferred_element_type=jnp.float32)

    kf = float(K)

    def select_slots(suh):
        lo = jnp.min(suh, axis=1, keepdims=True)
        hi = jnp.max(suh, axis=1, keepdims=True) + 1
        for _ in range(33):
            d = hi - lo
            mid = lo + jax.lax.shift_right_logical(d, 1)
            cnt = jnp.sum(jnp.where(suh >= mid, 1.0, 0.0), axis=1,
                          keepdims=True)
            pred = cnt >= kf
            lo = jnp.where(pred, mid, lo)
            hi = jnp.where(pred, hi, mid)
        strict = suh > lo
        tie = suh == lo
        ns = jnp.sum(jnp.where(strict, 1.0, 0.0), axis=1, keepdims=True)
        lim = kf - ns
        v = jnp.where(tie, 2048.0, jnp.where(strict, 1.0, 0.0))
        cumv = jax.lax.dot_general(v, ut_ref[...], (((1,), (0,)), ((), ())),
                                   preferred_element_type=jnp.float32)
        cumt = jnp.floor(cumv * (1.0 / 2048.0))
        cums = cumv - 2048.0 * cumt
        selected = strict | (tie & (cumt <= lim))
        pos = cums + jnp.minimum(cumt, lim)
        return jnp.where(selected, pos, 0.0)

    def gather_conv(posm, qh):
        acc = None
        for s in range(1, K + 1):
            oh = jnp.where(posm == float(s), 1.0, 0.0)
            h = jax.lax.dot_general(oh, p, (((1,), (0,)), ((), ())),
                                    preferred_element_type=jnp.float32) + qh
            h = _lrelu(h)
            if has_wb:
                h = _lrelu(jax.lax.dot_general(
                    h, wb_ref[...], (((1,), (0,)), ((), ())),
                    preferred_element_type=jnp.float32))
            acc = h if acc is None else jnp.maximum(acc, h)
        return acc

    hb = RB // 2
    posm = jnp.concatenate(
        [select_slots(su_ref[half * hb:(half + 1) * hb, :])
         for half in range(2)], axis=0)
    out_ref[0] = gather_conv(posm, q)


def _edge(xt, Wa, Wb):
    c = xt.shape[-1]
    ctrue = Wa.shape[1] // 2
    h = Wa.shape[0]
    wd = jnp.zeros((c, h), jnp.float32).at[:ctrue].set(Wa[:, :ctrue].T * BNS)
    wcd = jnp.zeros((c, h), jnp.float32).at[:ctrue].set(
        (Wa[:, ctrue:] - Wa[:, :ctrue]).T * BNS)
    has_wb = Wb is not None
    wbt = (Wb.T * BNS) if has_wb else jnp.zeros((h, h), jnp.float32)
    h2 = Wb.shape[0] if has_wb else h
    return pl.pallas_call(
        functools.partial(_edge_body, has_wb=has_wb),
        grid=(B, N // RB),
        in_specs=[
            pl.BlockSpec((1, N, c), lambda b, r: (b, 0, 0)),
            pl.BlockSpec(wd.shape, lambda b, r: (0, 0)),
            pl.BlockSpec(wcd.shape, lambda b, r: (0, 0)),
            pl.BlockSpec(wbt.shape, lambda b, r: (0, 0)),
        ],
        out_specs=pl.BlockSpec((1, RB, h2), lambda b, r: (b, r, 0)),
        out_shape=jax.ShapeDtypeStruct((B, N, h2), jnp.float32),
        scratch_shapes=[pltpu.VMEM((RB, N), jnp.int32),
                        pltpu.VMEM((N, N), jnp.float32)],
    )(xt, wd, wcd, wbt)


def _tnet_body(ha_ref, xt0_ref, wt3_ref, wl1_ref, wl2_ref, wtr_ref, btr_ref,
               out_ref):
    ha = ha_ref[0]
    h = _lrelu(jax.lax.dot_general(ha, wt3_ref[...], (((1,), (0,)), ((), ())),
                                   preferred_element_type=jnp.float32))
    h = jnp.max(h, axis=0, keepdims=True)
    h = _lrelu(jax.lax.dot_general(h, wl1_ref[...], (((1,), (0,)), ((), ())),
                                   preferred_element_type=jnp.float32))
    h = _lrelu(jax.lax.dot_general(h, wl2_ref[...], (((1,), (0,)), ((), ())),
                                   preferred_element_type=jnp.float32))
    t9 = jax.lax.dot_general(h, wtr_ref[...], (((1,), (0,)), ((), ())),
                             preferred_element_type=jnp.float32)
    t9 = t9 + btr_ref[...]
    er = jax.lax.broadcasted_iota(jnp.int32, (8, 16), 1)
    cr = jax.lax.broadcasted_iota(jnp.int32, (8, 16), 0)
    a = (er // 3 == cr).astype(jnp.float32)
    ec = jax.lax.broadcasted_iota(jnp.int32, (16, 8), 0)
    dc = jax.lax.broadcasted_iota(jnp.int32, (16, 8), 1)
    bmat = (ec % 3 == dc).astype(jnp.float32)
    diag = jnp.broadcast_to(t9, (16, 16)) * jnp.eye(16, dtype=jnp.float32)
    t8 = jax.lax.dot_general(
        jax.lax.dot_general(a, diag, (((1,), (0,)), ((), ())),
                            preferred_element_type=jnp.float32),
        bmat, (((1,), (0,)), ((), ())), preferred_element_type=jnp.float32)
    out_ref[0] = jax.lax.dot_general(
        xt0_ref[0], t8, (((1,), (0,)), ((), ())),
        preferred_element_type=jnp.float32)


def _tnet_tail(ha, xt0p, W_t3, W_l1, W_l2, W_tr, b_tr):
    wtr = jnp.zeros((256, 16), jnp.float32).at[:, :9].set(W_tr.T)
    btr = jnp.zeros((1, 16), jnp.float32).at[0, :9].set(b_tr)
    return pl.pallas_call(
        _tnet_body,
        grid=(B,),
        in_specs=[
            pl.BlockSpec((1, N, 128), lambda b: (b, 0, 0)),
            pl.BlockSpec((1, N, 8), lambda b: (b, 0, 0)),
            pl.BlockSpec((128, 1024), lambda b: (0, 0)),
            pl.BlockSpec((1024, 512), lambda b: (0, 0)),
            pl.BlockSpec((512, 256), lambda b: (0, 0)),
            pl.BlockSpec((256, 16), lambda b: (0, 0)),
            pl.BlockSpec((1, 16), lambda b: (0, 0)),
        ],
        out_specs=pl.BlockSpec((1, N, 8), lambda b: (b, 0, 0)),
        out_shape=jax.ShapeDtypeStruct((B, N, 8), jnp.float32),
    )(ha, xt0p, W_t3.T * BNS, W_l1.T * BNS, W_l2.T * BNS, wtr, btr)


def _tail_body(x1_ref, x2_ref, x3_ref, wm1_ref, w2ag_ref, w2ac_ref, w2b_ref,
               w2c_ref, b2c_ref, lat_ref):
    cat = jnp.concatenate([x1_ref[0], x2_ref[0], x3_ref[0]], axis=1)
    u = _lrelu(jax.lax.dot_general(cat, wm1_ref[...], (((1,), (0,)), ((), ())),
                                   preferred_element_type=jnp.float32))
    g = jnp.max(u, axis=0, keepdims=True)
    gp = jax.lax.dot_general(g, w2ag_ref[...], (((1,), (0,)), ((), ())),
                             preferred_element_type=jnp.float32)
    h = _lrelu(jax.lax.dot_general(cat, w2ac_ref[...], (((1,), (0,)), ((), ())),
                                   preferred_element_type=jnp.float32) + gp)
    h = _lrelu(jax.lax.dot_general(h, w2b_ref[...], (((1,), (0,)), ((), ())),
                                   preferred_element_type=jnp.float32))
    h = jax.lax.dot_general(h, w2c_ref[...], (((1,), (0,)), ((), ())),
                            preferred_element_type=jnp.float32) + b2c_ref[...]
    lat_ref[0] = jnp.max(h, axis=0, keepdims=True)


def _tail(x1, x2, x3, Wm1, Wm2a, Wm2b, Wm2c, bm2c):
    return pl.pallas_call(
        _tail_body,
        grid=(B,),
        in_specs=[
            pl.BlockSpec((1, N, 64), lambda b: (b, 0, 0)),
            pl.BlockSpec((1, N, 64), lambda b: (b, 0, 0)),
            pl.BlockSpec((1, N, 64), lambda b: (b, 0, 0)),
            pl.BlockSpec((192, 1024), lambda b: (0, 0)),
            pl.BlockSpec((1024, 1024), lambda b: (0, 0)),
            pl.BlockSpec((192, 1024), lambda b: (0, 0)),
            pl.BlockSpec((1024, 256), lambda b: (0, 0)),
            pl.BlockSpec((256, 256), lambda b: (0, 0)),
            pl.BlockSpec((1, 256), lambda b: (0, 0)),
        ],
        out_specs=pl.BlockSpec((1, 1, 256), lambda b: (b, 0, 0)),
        out_shape=jax.ShapeDtypeStruct((B, 1, 256), jnp.float32),
    )(x1, x2, x3, Wm1.T * BNS, Wm2a[:, :1024].T * BNS, Wm2a[:, 1024:].T * BNS,
      Wm2b.T * BNS, Wm2c.T, bm2c[None, :]).reshape(B, 256)


def _head_kernel(latent_ref, wm1_ref, bm1_ref, wm2_ref, bm2_ref,
                 wl1_ref, bl1_ref, wl2_ref, bl2_ref, e_ref,
                 z_ref, mean_ref, lv_ref):
    latent = latent_ref[...]
    m = jnp.maximum(latent @ wm1_ref[...].T * BNS + bm1_ref[...][None, :], 0.0)
    mean = m @ wm2_ref[...].T + bm2_ref[...][None, :]
    lv = jnp.maximum(latent @ wl1_ref[...].T * BNS + bl1_ref[...][None, :], 0.0)
    log_var = lv @ wl2_ref[...].T + bl2_ref[...][None, :]
    std = jnp.exp(0.5 * log_var)
    z_ref[...] = std * e_ref[...] + mean
    mean_ref[...] = mean
    lv_ref[...] = log_var


def _head(latent, Wmean1, bmean1, Wmean2, bmean2, Wlv1, blv1, Wlv2, blv2):
    e = jax.random.normal(jax.random.key(42), (B, 256), dtype=jnp.float32)
    return pl.pallas_call(
        _head_kernel,
        out_shape=(
            jax.ShapeDtypeStruct((B, 256), jnp.float32),
            jax.ShapeDtypeStruct((B, 256), jnp.float32),
            jax.ShapeDtypeStruct((B, 256), jnp.float32),
        ),
    )(latent, Wmean1, bmean1 * BNS, Wmean2, bmean2, Wlv1, blv1 * BNS,
      Wlv2, blv2, e)


def _graph_feature(x, idx):
    b, c, n = x.shape
    xt = jnp.transpose(x, (0, 2, 1))
    bidx = jnp.arange(b)[:, None, None]
    feat = xt[bidx, idx]
    xc = jnp.broadcast_to(xt[:, :, None, :], (b, n, K, c))
    out = jnp.concatenate([feat - xc, xc], axis=3)
    return jnp.transpose(out, (0, 3, 1, 2))


def kernel(x, W_t1, W_t2, W_t3, W_l1, W_l2, W_tr, b_tr, W1a, W1b, W2a, W2b,
           W3, Wm1, Wm2a, Wm2b, Wm2c, bm2c, Wmean1, bmean1, Wmean2, bmean2,
           Wlv1, blv1, Wlv2, blv2):
    def conv2d(W, f):
        return jnp.einsum('oc,bcnk->bonk', W, f)

    def conv1d(W, f):
        return jnp.einsum('oc,bcn->bon', W, f)

    xt0 = jnp.transpose(x, (0, 2, 1))
    xt0p = jnp.pad(xt0, ((0, 0), (0, 0), (0, 5)))
    ha = _edge(xt0p, W_t1, W_t2)
    xtrp = _tnet_tail(ha, xt0p, W_t3, W_l1, W_l2, W_tr, b_tr)

    x1 = _edge(xtrp, W1a, W1b)
    x2 = _edge(x1, W2a, W2b)
    x3 = _edge(x2, W3, None)
    latent = _tail(x1, x2, x3, Wm1, Wm2a, Wm2b, Wm2c, bm2c)
    return _head(latent, Wmean1 * BNS, bmean1, Wmean2, bmean2,
                 Wlv1 * BNS, blv1, Wlv2, blv2)


# speedup vs baseline: 1.1330x; 1.1330x over previous
import functools

import jax
import jax.numpy as jnp
from jax.experimental import pallas as pl
from jax.experimental.pallas import tpu as pltpu

B, N, K = 8, 1024, 40
EPS = 1e-5
BNS = 1.0 / (1.0 + EPS) ** 0.5
RB = 256
NEG = -jnp.inf


def _lrelu(x):
    return jnp.where(x > 0, x, 0.2 * x)


def _knn_body(xt_ref, idx_ref, pd_ref):
    r = pl.program_id(1)
    xt = xt_ref[0]
    rows = xt_ref[0, pl.ds(r * RB, RB), :]
    g = jax.lax.dot_general(rows, xt, (((1,), (1,)), ((), ())),
                            preferred_element_type=jnp.float32)
    xxc = jnp.sum(xt * xt, axis=1)[None, :]
    xxr = jnp.sum(rows * rows, axis=1)[:, None]
    pd_ref[...] = 2.0 * g - xxr - xxc

    iota = jax.lax.broadcasted_iota(jnp.int32, (RB, N), 1)

    sels = []
    for _ in range(K):
        pd = pd_ref[...]
        m = jnp.max(pd, axis=1, keepdims=True)
        cand = jnp.where(pd == m, iota, N)
        sel = jnp.min(cand, axis=1, keepdims=True)
        sels.append(sel)
        pd_ref[...] = jnp.where(iota == sel, NEG, pd)
    idx_ref[0] = jnp.concatenate(sels, axis=1)


def _knn_idx(xt):
    c = xt.shape[-1]
    return pl.pallas_call(
        _knn_body,
        grid=(B, N // RB),
        in_specs=[pl.BlockSpec((1, N, c), lambda b, r: (b, 0, 0))],
        out_specs=pl.BlockSpec((1, RB, K), lambda b, r: (b, r, 0)),
        out_shape=jax.ShapeDtypeStruct((B, N, K), jnp.int32),
        scratch_shapes=[pltpu.VMEM((RB, N), jnp.float32)],
    )(xt)


def _edge_body(xt_ref, wd_ref, wcd_ref, wb_ref, out_ref, pd_ref, *, has_wb):
    r = pl.program_id(1)
    xt = xt_ref[0]
    rows = xt_ref[0, pl.ds(r * RB, RB), :]
    g = jax.lax.dot_general(rows, xt, (((1,), (1,)), ((), ())),
                            preferred_element_type=jnp.float32)
    xxc = jnp.sum(xt * xt, axis=1)[None, :]
    xxr = jnp.sum(rows * rows, axis=1)[:, None]
    pd_ref[...] = 2.0 * g - xxr - xxc

    p = jax.lax.dot_general(xt, wd_ref[...], (((1,), (0,)), ((), ())),
                            preferred_element_type=jnp.float32)
    q = jax.lax.dot_general(rows, wcd_ref[...], (((1,), (0,)), ((), ())),
                            preferred_element_type=jnp.float32)

    iota = jax.lax.broadcasted_iota(jnp.int32, (RB, N), 1).astype(jnp.float32)
    acc = None
    for _ in range(K):
        pd = pd_ref[...]
        m = jnp.max(pd, axis=1, keepdims=True)
        cand = jnp.where(pd == m, iota, float(N))
        sel = jnp.min(cand, axis=1, keepdims=True)
        onehot = (iota == sel)
        pd_ref[...] = jnp.where(onehot, NEG, pd)
        oh = onehot.astype(jnp.float32)
        h = jax.lax.dot_general(oh, p, (((1,), (0,)), ((), ())),
                                preferred_element_type=jnp.float32) + q
        h = _lrelu(h)
        if has_wb:
            h = _lrelu(jax.lax.dot_general(
                h, wb_ref[...], (((1,), (0,)), ((), ())),
                preferred_element_type=jnp.float32))
        acc = h if acc is None else jnp.maximum(acc, h)
    out_ref[0] = acc


def _edge(xt, Wa, Wb):
    c = xt.shape[-1]
    ctrue = Wa.shape[1] // 2
    h = Wa.shape[0]
    wd = jnp.zeros((c, h), jnp.float32).at[:ctrue].set(Wa[:, :ctrue].T * BNS)
    wcd = jnp.zeros((c, h), jnp.float32).at[:ctrue].set(
        (Wa[:, ctrue:] - Wa[:, :ctrue]).T * BNS)
    has_wb = Wb is not None
    wbt = (Wb.T * BNS) if has_wb else jnp.zeros((h, h), jnp.float32)
    h2 = Wb.shape[0] if has_wb else h
    return pl.pallas_call(
        functools.partial(_edge_body, has_wb=has_wb),
        grid=(B, N // RB),
        in_specs=[
            pl.BlockSpec((1, N, c), lambda b, r: (b, 0, 0)),
            pl.BlockSpec(wd.shape, lambda b, r: (0, 0)),
            pl.BlockSpec(wcd.shape, lambda b, r: (0, 0)),
            pl.BlockSpec(wbt.shape, lambda b, r: (0, 0)),
        ],
        out_specs=pl.BlockSpec((1, RB, h2), lambda b, r: (b, r, 0)),
        out_shape=jax.ShapeDtypeStruct((B, N, h2), jnp.float32),
        scratch_shapes=[pltpu.VMEM((RB, N), jnp.float32)],
    )(xt, wd, wcd, wbt)


def _tnet_body(ha_ref, xt0_ref, wt3_ref, wl1_ref, wl2_ref, wtr_ref, btr_ref,
               out_ref):
    ha = ha_ref[0]
    h = _lrelu(jax.lax.dot_general(ha, wt3_ref[...], (((1,), (0,)), ((), ())),
                                   preferred_element_type=jnp.float32))
    h = jnp.max(h, axis=0, keepdims=True)
    h = _lrelu(jax.lax.dot_general(h, wl1_ref[...], (((1,), (0,)), ((), ())),
                                   preferred_element_type=jnp.float32))
    h = _lrelu(jax.lax.dot_general(h, wl2_ref[...], (((1,), (0,)), ((), ())),
                                   preferred_element_type=jnp.float32))
    t9 = jax.lax.dot_general(h, wtr_ref[...], (((1,), (0,)), ((), ())),
                             preferred_element_type=jnp.float32)
    t9 = t9 + btr_ref[...]
    er = jax.lax.broadcasted_iota(jnp.int32, (8, 16), 1)
    cr = jax.lax.broadcasted_iota(jnp.int32, (8, 16), 0)
    a = (er // 3 == cr).astype(jnp.float32)
    ec = jax.lax.broadcasted_iota(jnp.int32, (16, 8), 0)
    dc = jax.lax.broadcasted_iota(jnp.int32, (16, 8), 1)
    bmat = (ec % 3 == dc).astype(jnp.float32)
    diag = jnp.broadcast_to(t9, (16, 16)) * jnp.eye(16, dtype=jnp.float32)
    t8 = jax.lax.dot_general(
        jax.lax.dot_general(a, diag, (((1,), (0,)), ((), ())),
                            preferred_element_type=jnp.float32),
        bmat, (((1,), (0,)), ((), ())), preferred_element_type=jnp.float32)
    out_ref[0] = jax.lax.dot_general(
        xt0_ref[0], t8, (((1,), (0,)), ((), ())),
        preferred_element_type=jnp.float32)


def _tnet_tail(ha, xt0p, W_t3, W_l1, W_l2, W_tr, b_tr):
    wtr = jnp.zeros((256, 16), jnp.float32).at[:, :9].set(W_tr.T)
    btr = jnp.zeros((1, 16), jnp.float32).at[0, :9].set(b_tr)
    return pl.pallas_call(
        _tnet_body,
        grid=(B,),
        in_specs=[
            pl.BlockSpec((1, N, 128), lambda b: (b, 0, 0)),
            pl.BlockSpec((1, N, 8), lambda b: (b, 0, 0)),
            pl.BlockSpec((128, 1024), lambda b: (0, 0)),
            pl.BlockSpec((1024, 512), lambda b: (0, 0)),
            pl.BlockSpec((512, 256), lambda b: (0, 0)),
            pl.BlockSpec((256, 16), lambda b: (0, 0)),
            pl.BlockSpec((1, 16), lambda b: (0, 0)),
        ],
        out_specs=pl.BlockSpec((1, N, 8), lambda b: (b, 0, 0)),
        out_shape=jax.ShapeDtypeStruct((B, N, 8), jnp.float32),
    )(ha, xt0p, W_t3.T * BNS, W_l1.T * BNS, W_l2.T * BNS, wtr, btr)


def _tail_body(x1_ref, x2_ref, x3_ref, wm1_ref, w2ag_ref, w2ac_ref, w2b_ref,
               w2c_ref, b2c_ref, lat_ref):
    cat = jnp.concatenate([x1_ref[0], x2_ref[0], x3_ref[0]], axis=1)
    u = _lrelu(jax.lax.dot_general(cat, wm1_ref[...], (((1,), (0,)), ((), ())),
                                   preferred_element_type=jnp.float32))
    g = jnp.max(u, axis=0, keepdims=True)
    gp = jax.lax.dot_general(g, w2ag_ref[...], (((1,), (0,)), ((), ())),
                             preferred_element_type=jnp.float32)
    h = _lrelu(jax.lax.dot_general(cat, w2ac_ref[...], (((1,), (0,)), ((), ())),
                                   preferred_element_type=jnp.float32) + gp)
    h = _lrelu(jax.lax.dot_general(h, w2b_ref[...], (((1,), (0,)), ((), ())),
                                   preferred_element_type=jnp.float32))
    h = jax.lax.dot_general(h, w2c_ref[...], (((1,), (0,)), ((), ())),
                            preferred_element_type=jnp.float32) + b2c_ref[...]
    lat_ref[0] = jnp.max(h, axis=0, keepdims=True)


def _tail(x1, x2, x3, Wm1, Wm2a, Wm2b, Wm2c, bm2c):
    return pl.pallas_call(
        _tail_body,
        grid=(B,),
        in_specs=[
            pl.BlockSpec((1, N, 64), lambda b: (b, 0, 0)),
            pl.BlockSpec((1, N, 64), lambda b: (b, 0, 0)),
            pl.BlockSpec((1, N, 64), lambda b: (b, 0, 0)),
            pl.BlockSpec((192, 1024), lambda b: (0, 0)),
            pl.BlockSpec((1024, 1024), lambda b: (0, 0)),
            pl.BlockSpec((192, 1024), lambda b: (0, 0)),
            pl.BlockSpec((1024, 256), lambda b: (0, 0)),
            pl.BlockSpec((256, 256), lambda b: (0, 0)),
            pl.BlockSpec((1, 256), lambda b: (0, 0)),
        ],
        out_specs=pl.BlockSpec((1, 1, 256), lambda b: (b, 0, 0)),
        out_shape=jax.ShapeDtypeStruct((B, 1, 256), jnp.float32),
    )(x1, x2, x3, Wm1.T * BNS, Wm2a[:, :1024].T * BNS, Wm2a[:, 1024:].T * BNS,
      Wm2b.T * BNS, Wm2c.T, bm2c[None, :]).reshape(B, 256)


def _head_kernel(latent_ref, wm1_ref, bm1_ref, wm2_ref, bm2_ref,
                 wl1_ref, bl1_ref, wl2_ref, bl2_ref, e_ref,
                 z_ref, mean_ref, lv_ref):
    latent = latent_ref[...]
    m = jnp.maximum(latent @ wm1_ref[...].T * BNS + bm1_ref[...][None, :], 0.0)
    mean = m @ wm2_ref[...].T + bm2_ref[...][None, :]
    lv = jnp.maximum(latent @ wl1_ref[...].T * BNS + bl1_ref[...][None, :], 0.0)
    log_var = lv @ wl2_ref[...].T + bl2_ref[...][None, :]
    std = jnp.exp(0.5 * log_var)
    z_ref[...] = std * e_ref[...] + mean
    mean_ref[...] = mean
    lv_ref[...] = log_var


def _head(latent, Wmean1, bmean1, Wmean2, bmean2, Wlv1, blv1, Wlv2, blv2):
    e = jax.random.normal(jax.random.key(42), (B, 256), dtype=jnp.float32)
    return pl.pallas_call(
        _head_kernel,
        out_shape=(
            jax.ShapeDtypeStruct((B, 256), jnp.float32),
            jax.ShapeDtypeStruct((B, 256), jnp.float32),
            jax.ShapeDtypeStruct((B, 256), jnp.float32),
        ),
    )(latent, Wmean1, bmean1 * BNS, Wmean2, bmean2, Wlv1, blv1 * BNS,
      Wlv2, blv2, e)


def _graph_feature(x, idx):
    b, c, n = x.shape
    xt = jnp.transpose(x, (0, 2, 1))
    bidx = jnp.arange(b)[:, None, None]
    feat = xt[bidx, idx]
    xc = jnp.broadcast_to(xt[:, :, None, :], (b, n, K, c))
    out = jnp.concatenate([feat - xc, xc], axis=3)
    return jnp.transpose(out, (0, 3, 1, 2))


def kernel(x, W_t1, W_t2, W_t3, W_l1, W_l2, W_tr, b_tr, W1a, W1b, W2a, W2b,
           W3, Wm1, Wm2a, Wm2b, Wm2c, bm2c, Wmean1, bmean1, Wmean2, bmean2,
           Wlv1, blv1, Wlv2, blv2):
    def conv2d(W, f):
        return jnp.einsum('oc,bcnk->bonk', W, f)

    def conv1d(W, f):
        return jnp.einsum('oc,bcn->bon', W, f)

    xt0 = jnp.transpose(x, (0, 2, 1))
    xt0p = jnp.pad(xt0, ((0, 0), (0, 0), (0, 5)))
    ha = _edge(xt0p, W_t1, W_t2)
    xtrp = _tnet_tail(ha, xt0p, W_t3, W_l1, W_l2, W_tr, b_tr)

    x1 = _edge(xtrp, W1a, W1b)
    x2 = _edge(x1, W2a, W2b)
    x3 = _edge(x2, W3, None)
    latent = _tail(x1, x2, x3, Wm1, Wm2a, Wm2b, Wm2c, bm2c)
    return _head(latent, Wmean1 * BNS, bmean1, Wmean2, bmean2,
                 Wlv1 * BNS, blv1, Wlv2, blv2)
